# Initial kernel scaffold; baseline (speedup 1.0000x reference)
import jax
import jax.numpy as jnp
from jax.experimental import pallas as pl


def kernel(x, hyperedge_index):
    raise NotImplementedError("write your pallas kernel here")


# speedup vs baseline: 3.2557x; 3.2557x over previous
import functools

import jax
import jax.numpy as jnp
from jax import lax
from jax.experimental import pallas as pl
from jax.experimental.pallas import tpu as pltpu
from jax.experimental.pallas import tpu_sc as plsc

K = 4
N_NODES = 10000
D_FEAT = 128
D_HALF = D_FEAT // 2
N_EDGES = 320000
NUM_SUBCORES = 16
CHUNK = 128
CHUNKS = -(-N_EDGES // NUM_SUBCORES // CHUNK)
EDGES_PAD = NUM_SUBCORES * CHUNKS * CHUNK
ROWS_PER_SUB = 624
ROWS_TAIL_AT = ROWS_PER_SUB * NUM_SUBCORES
ROWS_TAIL = N_NODES - ROWS_TAIL_AT


@functools.partial(
    pl.kernel,
    out_type=[
        jax.ShapeDtypeStruct((N_NODES, D_HALF), jnp.float32),
        jax.ShapeDtypeStruct((N_NODES, D_HALF), jnp.float32),
    ],
    mesh=plsc.VectorSubcoreMesh(core_axis_name="c", subcore_axis_name="s"),
    scratch_types=[
        pltpu.VMEM((CHUNKS, CHUNK), jnp.int32),
        pltpu.VMEM((CHUNKS, CHUNK), jnp.int32),
        pltpu.VMEM((2, CHUNK, D_HALF), jnp.float32),
        pltpu.VMEM_SHARED((N_NODES + 1, D_HALF), jnp.float32),
        pltpu.SemaphoreType.DMA,
    ],
    compiler_params=pltpu.CompilerParams(use_tc_tiling_on_sc=False),
)
def _hgnn_sc(xa, xb, src_idx, dst_idx, out_a, out_b,
             src_v, dst_v, rows_v, acc, gsem):
    cid = lax.axis_index("c")
    sid = lax.axis_index("s")

    def copy_rows(src, dst):
        r0 = sid * ROWS_PER_SUB
        pltpu.sync_copy(src.at[pl.ds(r0, ROWS_PER_SUB)],
                        dst.at[pl.ds(r0, ROWS_PER_SUB)])

        @pl.when(sid == NUM_SUBCORES - 1)
        def _():
            pltpu.sync_copy(src.at[pl.ds(ROWS_TAIL_AT, ROWS_TAIL)],
                            dst.at[pl.ds(ROWS_TAIL_AT, ROWS_TAIL)])

    def run(table, out):
        copy_rows(table, acc)
        pltpu.sync_copy(src_idx.at[sid], src_v)
        pltpu.sync_copy(dst_idx.at[sid], dst_v)
        plsc.subcore_barrier()

        pltpu.async_copy(table.at[src_v.at[0]], rows_v.at[0], gsem)

        def step(j, carry):
            cur = lax.rem(j, 2)
            pltpu.make_async_copy(table.at[src_v.at[j]], rows_v.at[cur],
                                  gsem).wait()

            @pl.when(j + 1 < CHUNKS)
            def _():
                pltpu.async_copy(table.at[src_v.at[j + 1]],
                                 rows_v.at[1 - cur], gsem)

            pltpu.sync_copy(rows_v.at[cur], acc.at[dst_v.at[j]], add=True)
            return carry

        lax.fori_loop(0, CHUNKS, step, 0)
        plsc.subcore_barrier()
        copy_rows(acc, out)

    @pl.when(cid == 0)
    def _():
        run(xa, out_a)

    @pl.when(cid == 1)
    def _():
        run(xb, out_b)


def kernel(x, hyperedge_index):
    src = hyperedge_index[0]
    dst = hyperedge_index[1].reshape(-1, K)[:, 0]
    dst4 = jnp.broadcast_to(dst[:, None], (N_EDGES // K, K)).reshape(-1)
    pad = EDGES_PAD - N_EDGES
    srcp = jnp.concatenate([src, jnp.zeros((pad,), jnp.int32)])
    dstp = jnp.concatenate([dst4, jnp.full((pad,), N_NODES, jnp.int32)])
    srcp = srcp.reshape(NUM_SUBCORES, CHUNKS, CHUNK)
    dstp = dstp.reshape(NUM_SUBCORES, CHUNKS, CHUNK)
    xa = x[:, :D_HALF]
    xb = x[:, D_HALF:]
    out_a, out_b = _hgnn_sc(xa, xb, srcp, dstp)
    return jnp.concatenate([out_a, out_b], axis=1)
